# baseline (speedup 1.0000x reference)
import jax
import jax.numpy as jnp
from jax import lax
from jax.experimental import pallas as pl
from jax.experimental.pallas import tpu as pltpu
from jax.experimental.pallas import tpu_sc as plsc

VOCAB = 100000
EMBED = 16
BATCH = 1024
BAG = 20

NUM_CORES = 2
NUM_SUBCORES = 16
NUM_WORKERS = NUM_CORES * NUM_SUBCORES
B_PER_W = BATCH // NUM_WORKERS

VT = 1408
NV = 71
TAIL = VOCAB - VT * NV
NSLOT = 8


def _pool_body(idx_hbm, table_hbm, out_hbm, idx_v, rows_v, pooled_v, sem):
    wid = lax.axis_index("s") * NUM_CORES + lax.axis_index("c")
    base = wid * B_PER_W
    pltpu.sync_copy(idx_hbm.at[pl.ds(base * BAG, B_PER_W * BAG)], idx_v)
    pltpu.async_copy(table_hbm.at[idx_v], rows_v, sem).wait()
    for i in range(B_PER_W):
        r = rows_v[i * BAG, :]
        for j in range(1, BAG):
            r = r + rows_v[i * BAG + j, :]
        pooled_v[i, :] = r * (1.0 / BAG)
    pltpu.sync_copy(pooled_v, out_hbm.at[pl.ds(base, B_PER_W)])


def _pool(idx_flat, emb_table):
    return pl.kernel(
        _pool_body,
        out_type=jax.ShapeDtypeStruct((BATCH, EMBED), jnp.float32),
        mesh=plsc.VectorSubcoreMesh(core_axis_name="c", subcore_axis_name="s"),
        scratch_types=[
            pltpu.VMEM((B_PER_W * BAG,), jnp.int32),
            pltpu.VMEM((B_PER_W * BAG, EMBED), jnp.float32),
            pltpu.VMEM((B_PER_W, EMBED), jnp.float32),
            pltpu.SemaphoreType.DMA,
        ],
        compiler_params=pltpu.CompilerParams(use_tc_tiling_on_sc=False),
    )(idx_flat, emb_table)


def _stripe_copy(acc, out_hbm, sems, slot, v):
    return pltpu.make_async_copy(
        acc.at[slot], out_hbm.at[:, pl.ds(v * VT, VT)], sems.at[slot])


def _proj_body(pooled_hbm, wt_hbm, b_hbm, out_hbm,
               pooled_v, wt_v, b_v, acc, acc_t, sems, sem_t, sem_in):
    in_cps = [
        pltpu.make_async_copy(pooled_hbm, pooled_v, sem_in.at[0]),
        pltpu.make_async_copy(wt_hbm, wt_v, sem_in.at[1]),
        pltpu.make_async_copy(b_hbm, b_v, sem_in.at[2]),
    ]
    for cp in in_cps:
        cp.start()
    for cp in in_cps:
        cp.wait()

    def step(v, carry):
        slot = lax.rem(v, NSLOT)
        col = pl.multiple_of(v * VT, 128)

        @pl.when(v >= NSLOT)
        def _():
            _stripe_copy(acc, out_hbm, sems, slot, v - NSLOT).wait()

        acc[slot] = (
            jnp.dot(pooled_v[...], wt_v[:, pl.ds(col, VT)],
                    preferred_element_type=jnp.float32)
            + b_v[:, pl.ds(col, VT)]
        )
        _stripe_copy(acc, out_hbm, sems, slot, v).start()
        return carry

    lax.fori_loop(0, NV, step, 0)

    acc_t[...] = (
        jnp.dot(pooled_v[...], wt_v[:, pl.ds(VT * NV, TAIL)],
                preferred_element_type=jnp.float32)
        + b_v[:, pl.ds(VT * NV, TAIL)]
    )
    tail_cp = pltpu.make_async_copy(
        acc_t, out_hbm.at[:, pl.ds(VT * NV, TAIL)], sem_t)
    tail_cp.start()
    for back in range(NSLOT):
        v = NV - 1 - back
        _stripe_copy(acc, out_hbm, sems, v % NSLOT, v).wait()
    tail_cp.wait()


_proj = pl.pallas_call(
    _proj_body,
    in_specs=[
        pl.BlockSpec(memory_space=pl.ANY),
        pl.BlockSpec(memory_space=pl.ANY),
        pl.BlockSpec(memory_space=pl.ANY),
    ],
    out_specs=pl.BlockSpec(memory_space=pl.ANY),
    out_shape=jax.ShapeDtypeStruct((BATCH, VOCAB), jnp.float32),
    scratch_shapes=[
        pltpu.VMEM((BATCH, EMBED), jnp.float32),
        pltpu.VMEM((EMBED, VOCAB), jnp.float32),
        pltpu.VMEM((1, VOCAB), jnp.float32),
        pltpu.VMEM((NSLOT, BATCH, VT), jnp.float32),
        pltpu.VMEM((BATCH, TAIL), jnp.float32),
        pltpu.SemaphoreType.DMA((NSLOT,)),
        pltpu.SemaphoreType.DMA,
        pltpu.SemaphoreType.DMA((3,)),
    ],
    compiler_params=pltpu.CompilerParams(
        vmem_limit_bytes=64 * 1024 * 1024,
    ),
)


def kernel(inputs, emb_table, W, b):
    idx_flat = inputs.reshape(-1).astype(jnp.int32)
    pooled = _pool(idx_flat, emb_table)
    return _proj(pooled, W.T, b.reshape(1, VOCAB))
